# baseline (speedup 1.0000x reference)
import functools

import jax
import jax.numpy as jnp
from jax.experimental import pallas as pl
from jax.experimental.pallas import tpu as pltpu

CB = 8192
DIM = 256
BETA = 0.25
TM = 1024
JB = 1024
NJ = CB // JB


def _dist_kernel(x_ref, a_ref, cb_ref, idx_ref,
                 s0_ref, s1_ref, u_ref, bestd_ref, besti_ref):
    j = pl.program_id(1)

    @pl.when(j == 0)
    def _init():
        u_ref[...] = a_ref[...] + float(DIM)
        bestd_ref[...] = jnp.full((TM, 1), jnp.inf, jnp.float32)
        besti_ref[...] = jnp.zeros((TM, 1), jnp.int32)

    def dot_into(s_ref):
        s_ref[...] = jax.lax.dot_general(
            x_ref[...], cb_ref[...],
            dimension_numbers=(((1,), (1,)), ((), ())),
            preferred_element_type=jnp.float32,
        )

    def epilogue(s_ref):
        jb = j - 1
        u = u_ref[...]
        s = s_ref[...]
        dmin = jnp.min(u - s, axis=1, keepdims=True)
        cols = jax.lax.broadcasted_iota(jnp.int32, (TM, JB), 1)
        cand = jnp.where((u - s) == dmin, cols, jnp.int32(JB))
        imin = jnp.min(cand, axis=1, keepdims=True) + jb * JB

        better = dmin < bestd_ref[...]
        tie = dmin == bestd_ref[...]
        besti_ref[...] = jnp.where(
            better, imin,
            jnp.where(tie, jnp.minimum(imin, besti_ref[...]), besti_ref[...]))
        bestd_ref[...] = jnp.where(better, dmin, bestd_ref[...])

        @pl.when(jb == NJ // 2 - 1)
        def _round():
            bestd_ref[...] = (
                bestd_ref[...].astype(jnp.bfloat16).astype(jnp.float32))

    even = j % 2 == 0

    @pl.when((j < NJ) & even)
    def _d0():
        dot_into(s0_ref)

    @pl.when((j < NJ) & jnp.logical_not(even))
    def _d1():
        dot_into(s1_ref)

    @pl.when((j > 0) & jnp.logical_not(even))
    def _e0():
        epilogue(s0_ref)

    @pl.when((j > 0) & even)
    def _e1():
        epilogue(s1_ref)

    @pl.when(j == NJ)
    def _out():
        idx_ref[...] = besti_ref[...]


def _argmin_indices(x2_bf, a, cb_bf):
    T = x2_bf.shape[0]
    return pl.pallas_call(
        _dist_kernel,
        grid=(T // TM, NJ + 1),
        in_specs=[
            pl.BlockSpec((TM, DIM), lambda t, j: (t, 0)),
            pl.BlockSpec((TM, 1), lambda t, j: (t, 0)),
            pl.BlockSpec((JB, DIM), lambda t, j: (jnp.minimum(j, NJ - 1), 0)),
        ],
        out_specs=pl.BlockSpec((TM, 1), lambda t, j: (t, 0)),
        out_shape=jax.ShapeDtypeStruct((T, 1), jnp.int32),
        scratch_shapes=[
            pltpu.VMEM((TM, JB), jnp.float32),
            pltpu.VMEM((TM, JB), jnp.float32),
            pltpu.VMEM((TM, 1), jnp.float32),
            pltpu.VMEM((TM, 1), jnp.float32),
            pltpu.VMEM((TM, 1), jnp.int32),
        ],
        compiler_params=pltpu.CompilerParams(
            dimension_semantics=("parallel", "arbitrary")),
    )(x2_bf, a, cb_bf)


def kernel(x, codebook):
    B, C, W, H = x.shape
    xq = jax.nn.sigmoid(x * 100.0)
    xq = xq * 2.0 - 1.0
    flat_x = jnp.transpose(xq, (0, 2, 3, 1)).reshape(-1, C)
    a = jnp.sum(flat_x ** 2, axis=1, keepdims=True)
    x2_bf = (flat_x * 2.0).astype(jnp.bfloat16)
    cb_bf = codebook.astype(jnp.bfloat16)
    indices = _argmin_indices(x2_bf, a, cb_bf).reshape(-1)

    n_tok = B * W * H
    counts = jnp.bincount(indices, length=CB)
    cf = counts.astype(jnp.float32)
    p = cf / float(n_tok)
    log_probs = jnp.log(jnp.maximum(cf, 1.0)) - jnp.log(float(n_tok))
    entropy = -jnp.sum(jnp.where(counts > 0, p * log_probs, 0.0))
    perplexity = jnp.exp(entropy)
    perplexity_loss = 1.0 / perplexity
    quantized = jnp.take(codebook, indices, axis=0).reshape(B, W, H, C)
    quantized = jnp.transpose(quantized, (0, 3, 1, 2))
    loss = BETA * perplexity_loss
    quantized = xq + jax.lax.stop_gradient(quantized - xq)
    return (quantized, perplexity_loss, loss)


# speedup vs baseline: 1.1019x; 1.1019x over previous
import functools

import jax
import jax.numpy as jnp
from jax.experimental import pallas as pl
from jax.experimental.pallas import tpu as pltpu

CB = 8192
DIM = 256
BETA = 0.25
TM = 1024
JB = 1024
NJ = CB // JB


def _dist_kernel(x_ref, a_ref, cb_ref, idx_ref,
                 s0_ref, s1_ref, u_ref, bestd_ref, besti_ref):
    j = pl.program_id(1)

    @pl.when(j == 0)
    def _init():
        u_ref[...] = a_ref[0] + float(DIM)
        bestd_ref[...] = jnp.full((1, TM), jnp.inf, jnp.float32)
        besti_ref[...] = jnp.zeros((1, TM), jnp.int32)
        s1_ref[...] = jnp.full((JB, TM), jnp.nan, jnp.float32)

    def dot_into(s_ref):
        s_ref[...] = jax.lax.dot_general(
            cb_ref[...], x_ref[0],
            dimension_numbers=(((1,), (0,)), ((), ())),
            preferred_element_type=jnp.float32,
        )

    def epilogue(s_ref):
        jb = j - 1
        u = u_ref[...]
        s = s_ref[...]
        dmin = jnp.min(u - s, axis=0, keepdims=True)
        rows = jax.lax.broadcasted_iota(jnp.int32, (JB, TM), 0)
        cand = jnp.where((u - s) == dmin, rows, jnp.int32(JB))
        imin = jnp.min(cand, axis=0, keepdims=True) + jb * JB

        better = dmin < bestd_ref[...]
        tie = dmin == bestd_ref[...]
        besti_ref[...] = jnp.where(
            better, imin,
            jnp.where(tie, jnp.minimum(imin, besti_ref[...]), besti_ref[...]))
        nbd = jnp.where(better, dmin, bestd_ref[...])
        bestd_ref[...] = jnp.where(
            jb == NJ // 2 - 1,
            nbd.astype(jnp.bfloat16).astype(jnp.float32), nbd)
        idx_ref[0] = besti_ref[...]

    even = j % 2 == 0

    @pl.when(even)
    def _even():
        dot_into(s0_ref)
        epilogue(s1_ref)

    @pl.when(jnp.logical_not(even))
    def _odd():
        dot_into(s1_ref)
        epilogue(s0_ref)


def _argmin_indices(x2_bf3, a3, cb_bf):
    B = x2_bf3.shape[0]
    return pl.pallas_call(
        _dist_kernel,
        grid=(B, NJ + 1),
        in_specs=[
            pl.BlockSpec((1, DIM, TM), lambda t, j: (t, 0, 0)),
            pl.BlockSpec((1, 1, TM), lambda t, j: (t, 0, 0)),
            pl.BlockSpec((JB, DIM), lambda t, j: (jnp.minimum(j, NJ - 1), 0)),
        ],
        out_specs=pl.BlockSpec((1, 1, TM), lambda t, j: (t, 0, 0)),
        out_shape=jax.ShapeDtypeStruct((B, 1, TM), jnp.int32),
        scratch_shapes=[
            pltpu.VMEM((JB, TM), jnp.float32),
            pltpu.VMEM((JB, TM), jnp.float32),
            pltpu.VMEM((1, TM), jnp.float32),
            pltpu.VMEM((1, TM), jnp.float32),
            pltpu.VMEM((1, TM), jnp.int32),
        ],
        compiler_params=pltpu.CompilerParams(
            dimension_semantics=("parallel", "arbitrary")),
    )(x2_bf3, a3, cb_bf)


def kernel(x, codebook):
    B, C, W, H = x.shape
    xq = jax.nn.sigmoid(x * 100.0)
    xq = xq * 2.0 - 1.0
    flat_x = jnp.transpose(xq, (0, 2, 3, 1)).reshape(-1, C)
    a = jnp.sum(flat_x ** 2, axis=1, keepdims=True)
    a3 = a.reshape(B, 1, W * H)
    x2_bf3 = (xq * 2.0).astype(jnp.bfloat16).reshape(B, C, W * H)
    cb_bf = codebook.astype(jnp.bfloat16)
    indices = _argmin_indices(x2_bf3, a3, cb_bf).reshape(-1)

    n_tok = B * W * H
    counts = jnp.bincount(indices, length=CB)
    cf = counts.astype(jnp.float32)
    p = cf / float(n_tok)
    log_probs = jnp.log(jnp.maximum(cf, 1.0)) - jnp.log(float(n_tok))
    entropy = -jnp.sum(jnp.where(counts > 0, p * log_probs, 0.0))
    perplexity = jnp.exp(entropy)
    perplexity_loss = 1.0 / perplexity
    quantized = jnp.take(codebook, indices, axis=0).reshape(B, W, H, C)
    quantized = jnp.transpose(quantized, (0, 3, 1, 2))
    loss = BETA * perplexity_loss
    quantized = xq + jax.lax.stop_gradient(quantized - xq)
    return (quantized, perplexity_loss, loss)


# speedup vs baseline: 1.1092x; 1.0066x over previous
import functools

import jax
import jax.numpy as jnp
from jax.experimental import pallas as pl
from jax.experimental.pallas import tpu as pltpu

CB = 8192
DIM = 256
BETA = 0.25
TM = 1024
JB = 1024
NJ = CB // JB


def _dist_kernel(x_ref, a_ref, cb_ref, idx_ref,
                 s0_ref, s1_ref, u_ref, bestd_ref, besti_ref):
    j = pl.program_id(1)

    @pl.when(j == 0)
    def _init():
        u_ref[...] = a_ref[0] + float(DIM)
        bestd_ref[...] = jnp.full((1, TM), jnp.inf, jnp.float32)
        besti_ref[...] = jnp.zeros((1, TM), jnp.int32)
        s1_ref[...] = jnp.full((JB, TM), jnp.nan, jnp.float32)

    HB = JB // 2

    def dot_into(s_ref):
        x = x_ref[0]
        for h in range(2):
            s_ref[pl.ds(h * HB, HB), :] = jax.lax.dot_general(
                cb_ref[pl.ds(h * HB, HB), :], x,
                dimension_numbers=(((1,), (0,)), ((), ())),
                preferred_element_type=jnp.float32,
            )

    def _merge(dmin, imin):
        better = dmin < bestd_ref[...]
        tie = dmin == bestd_ref[...]
        besti_ref[...] = jnp.where(
            better, imin,
            jnp.where(tie, jnp.minimum(imin, besti_ref[...]), besti_ref[...]))
        bestd_ref[...] = jnp.where(better, dmin, bestd_ref[...])

    def epilogue(s_ref):
        jb = j - 1
        u = u_ref[...]
        rows = jax.lax.broadcasted_iota(jnp.int32, (HB, TM), 0)
        for h in range(2):
            s = s_ref[pl.ds(h * HB, HB), :]
            d = u - s
            dmin = jnp.min(d, axis=0, keepdims=True)
            cand = jnp.where(d == dmin, rows, jnp.int32(JB))
            imin = jnp.min(cand, axis=0, keepdims=True) + (jb * JB + h * HB)
            _merge(dmin, imin)
        bestd_ref[...] = jnp.where(
            jb == NJ // 2 - 1,
            bestd_ref[...].astype(jnp.bfloat16).astype(jnp.float32),
            bestd_ref[...])
        idx_ref[0] = besti_ref[...]

    even = j % 2 == 0

    @pl.when(even)
    def _even():
        dot_into(s0_ref)
        epilogue(s1_ref)

    @pl.when(jnp.logical_not(even))
    def _odd():
        dot_into(s1_ref)
        epilogue(s0_ref)


def _argmin_indices(x2_bf3, a3, cb_bf):
    B = x2_bf3.shape[0]
    return pl.pallas_call(
        _dist_kernel,
        grid=(B, NJ + 1),
        in_specs=[
            pl.BlockSpec((1, DIM, TM), lambda t, j: (t, 0, 0)),
            pl.BlockSpec((1, 1, TM), lambda t, j: (t, 0, 0)),
            pl.BlockSpec((JB, DIM), lambda t, j: (jnp.minimum(j, NJ - 1), 0)),
        ],
        out_specs=pl.BlockSpec((1, 1, TM), lambda t, j: (t, 0, 0)),
        out_shape=jax.ShapeDtypeStruct((B, 1, TM), jnp.int32),
        scratch_shapes=[
            pltpu.VMEM((JB, TM), jnp.float32),
            pltpu.VMEM((JB, TM), jnp.float32),
            pltpu.VMEM((1, TM), jnp.float32),
            pltpu.VMEM((1, TM), jnp.float32),
            pltpu.VMEM((1, TM), jnp.int32),
        ],
        compiler_params=pltpu.CompilerParams(
            dimension_semantics=("parallel", "arbitrary")),
    )(x2_bf3, a3, cb_bf)


def kernel(x, codebook):
    B, C, W, H = x.shape
    xq = jax.nn.sigmoid(x * 100.0)
    xq = xq * 2.0 - 1.0
    flat_x = jnp.transpose(xq, (0, 2, 3, 1)).reshape(-1, C)
    a = jnp.sum(flat_x ** 2, axis=1, keepdims=True)
    a3 = a.reshape(B, 1, W * H)
    x2_bf3 = (xq * 2.0).astype(jnp.bfloat16).reshape(B, C, W * H)
    cb_bf = codebook.astype(jnp.bfloat16)
    indices = _argmin_indices(x2_bf3, a3, cb_bf).reshape(-1)

    n_tok = B * W * H
    counts = jnp.bincount(indices, length=CB)
    cf = counts.astype(jnp.float32)
    p = cf / float(n_tok)
    log_probs = jnp.log(jnp.maximum(cf, 1.0)) - jnp.log(float(n_tok))
    entropy = -jnp.sum(jnp.where(counts > 0, p * log_probs, 0.0))
    perplexity = jnp.exp(entropy)
    perplexity_loss = 1.0 / perplexity
    quantized = jnp.take(codebook, indices, axis=0).reshape(B, W, H, C)
    quantized = jnp.transpose(quantized, (0, 3, 1, 2))
    loss = BETA * perplexity_loss
    quantized = xq + jax.lax.stop_gradient(quantized - xq)
    return (quantized, perplexity_loss, loss)


# speedup vs baseline: 1.1244x; 1.0137x over previous
import functools

import jax
import jax.numpy as jnp
from jax.experimental import pallas as pl
from jax.experimental.pallas import tpu as pltpu

CB = 8192
DIM = 256
BETA = 0.25
IM = 1024
NI = 2
TM = IM * NI
JB = 1024
NJ = CB // JB


def _dist_kernel(x_ref, a_ref, cb_ref, idx_ref,
                 s0_ref, s1_ref, u_ref, bestd_ref, besti_ref):
    j = pl.program_id(1)

    @pl.when(j == 0)
    def _init():
        for i in range(NI):
            u_ref[:, pl.ds(i * IM, IM)] = a_ref[i] + float(DIM)
        bestd_ref[...] = jnp.full((1, TM), jnp.inf, jnp.float32)
        besti_ref[...] = jnp.zeros((1, TM), jnp.int32)
        s1_ref[...] = jnp.full((JB, TM), jnp.nan, jnp.float32)

    def dot_into(s_ref):
        for i in range(NI):
            s_ref[:, pl.ds(i * IM, IM)] = jax.lax.dot_general(
                cb_ref[...], x_ref[i],
                dimension_numbers=(((1,), (0,)), ((), ())),
                preferred_element_type=jnp.float32,
            )

    def epilogue(s_ref):
        jb = j - 1
        u = u_ref[...]
        s = s_ref[...]
        d = u - s
        dmin = jnp.min(d, axis=0, keepdims=True)
        rows = jax.lax.broadcasted_iota(jnp.int32, (JB, TM), 0)
        cand = jnp.where(d == dmin, rows, jnp.int32(JB))
        imin = jnp.min(cand, axis=0, keepdims=True) + jb * JB

        better = dmin < bestd_ref[...]
        tie = dmin == bestd_ref[...]
        besti_ref[...] = jnp.where(
            better, imin,
            jnp.where(tie, jnp.minimum(imin, besti_ref[...]), besti_ref[...]))
        nbd = jnp.where(better, dmin, bestd_ref[...])
        bestd_ref[...] = jnp.where(
            jb == NJ // 2 - 1,
            nbd.astype(jnp.bfloat16).astype(jnp.float32), nbd)
        for i in range(NI):
            idx_ref[i] = besti_ref[:, pl.ds(i * IM, IM)]

    even = j % 2 == 0

    @pl.when(even)
    def _even():
        dot_into(s0_ref)
        epilogue(s1_ref)

    @pl.when(jnp.logical_not(even))
    def _odd():
        dot_into(s1_ref)
        epilogue(s0_ref)


def _argmin_indices(x2_bf3, a3, cb_bf):
    B = x2_bf3.shape[0]
    return pl.pallas_call(
        _dist_kernel,
        grid=(B // NI, NJ + 1),
        in_specs=[
            pl.BlockSpec((NI, DIM, IM), lambda t, j: (t, 0, 0)),
            pl.BlockSpec((NI, 1, IM), lambda t, j: (t, 0, 0)),
            pl.BlockSpec((JB, DIM), lambda t, j: (jnp.minimum(j, NJ - 1), 0)),
        ],
        out_specs=pl.BlockSpec((NI, 1, IM), lambda t, j: (t, 0, 0)),
        out_shape=jax.ShapeDtypeStruct((B, 1, IM), jnp.int32),
        scratch_shapes=[
            pltpu.VMEM((JB, TM), jnp.float32),
            pltpu.VMEM((JB, TM), jnp.float32),
            pltpu.VMEM((1, TM), jnp.float32),
            pltpu.VMEM((1, TM), jnp.float32),
            pltpu.VMEM((1, TM), jnp.int32),
        ],
        compiler_params=pltpu.CompilerParams(
            dimension_semantics=("parallel", "arbitrary")),
    )(x2_bf3, a3, cb_bf)


def kernel(x, codebook):
    B, C, W, H = x.shape
    xq = jax.nn.sigmoid(x * 100.0)
    xq = xq * 2.0 - 1.0
    flat_x = jnp.transpose(xq, (0, 2, 3, 1)).reshape(-1, C)
    a = jnp.sum(flat_x ** 2, axis=1, keepdims=True)
    a3 = a.reshape(B, 1, W * H)
    x2_bf3 = (xq * 2.0).astype(jnp.bfloat16).reshape(B, C, W * H)
    cb_bf = codebook.astype(jnp.bfloat16)
    indices = _argmin_indices(x2_bf3, a3, cb_bf).reshape(-1)

    n_tok = B * W * H
    counts = jnp.bincount(indices, length=CB)
    cf = counts.astype(jnp.float32)
    p = cf / float(n_tok)
    log_probs = jnp.log(jnp.maximum(cf, 1.0)) - jnp.log(float(n_tok))
    entropy = -jnp.sum(jnp.where(counts > 0, p * log_probs, 0.0))
    perplexity = jnp.exp(entropy)
    perplexity_loss = 1.0 / perplexity
    quantized = jnp.take(codebook, indices, axis=0).reshape(B, W, H, C)
    quantized = jnp.transpose(quantized, (0, 3, 1, 2))
    loss = BETA * perplexity_loss
    quantized = xq + jax.lax.stop_gradient(quantized - xq)
    return (quantized, perplexity_loss, loss)


# speedup vs baseline: 1.1368x; 1.0111x over previous
import functools

import jax
import jax.numpy as jnp
from jax.experimental import pallas as pl
from jax.experimental.pallas import tpu as pltpu

CB = 8192
DIM = 256
BETA = 0.25
TM = 1024
JB = 1024
NJ = CB // JB


def _dist_kernel(x_ref, a_ref, cb_ref, idx_ref, bestd_ref, besti_ref):
    j = pl.program_id(1)

    @pl.when(j == 0)
    def _init():
        bestd_ref[...] = jnp.full((TM, 1), jnp.inf, jnp.float32)
        besti_ref[...] = jnp.zeros((TM, 1), jnp.int32)

    s = jax.lax.dot_general(
        x_ref[...], cb_ref[...],
        dimension_numbers=(((1,), (1,)), ((), ())),
        preferred_element_type=jnp.float32,
    )
    u = a_ref[...] + float(DIM)
    d = u - s
    dmin = jnp.min(d, axis=1, keepdims=True)
    cols = jax.lax.broadcasted_iota(jnp.int32, (TM, JB), 1)
    cand = jnp.where(d == dmin, cols, jnp.int32(JB))
    imin = jnp.min(cand, axis=1, keepdims=True) + j * JB

    better = dmin < bestd_ref[...]
    tie = dmin == bestd_ref[...]
    besti_ref[...] = jnp.where(
        better, imin,
        jnp.where(tie, jnp.minimum(imin, besti_ref[...]), besti_ref[...]))
    nbd = jnp.where(better, dmin, bestd_ref[...])
    bestd_ref[...] = jnp.where(
        j == NJ // 2 - 1,
        nbd.astype(jnp.bfloat16).astype(jnp.float32), nbd)

    @pl.when(j == NJ - 1)
    def _out():
        idx_ref[...] = besti_ref[...]


def _argmin_indices(x2_bf, a, cb_bf):
    T = x2_bf.shape[0]
    return pl.pallas_call(
        _dist_kernel,
        grid=(T // TM, NJ),
        in_specs=[
            pl.BlockSpec((TM, DIM), lambda t, j: (t, 0)),
            pl.BlockSpec((TM, 1), lambda t, j: (t, 0)),
            pl.BlockSpec((JB, DIM), lambda t, j: (j, 0)),
        ],
        out_specs=pl.BlockSpec((TM, 1), lambda t, j: (t, 0)),
        out_shape=jax.ShapeDtypeStruct((T, 1), jnp.int32),
        scratch_shapes=[
            pltpu.VMEM((TM, 1), jnp.float32),
            pltpu.VMEM((TM, 1), jnp.int32),
        ],
        compiler_params=pltpu.CompilerParams(
            dimension_semantics=("parallel", "arbitrary")),
    )(x2_bf, a, cb_bf)


def kernel(x, codebook):
    B, C, W, H = x.shape
    xq = jax.nn.sigmoid(x * 100.0)
    xq = xq * 2.0 - 1.0
    flat_x = jnp.transpose(xq, (0, 2, 3, 1)).reshape(-1, C)
    a = jnp.sum(flat_x ** 2, axis=1, keepdims=True)
    x2_bf = (flat_x * 2.0).astype(jnp.bfloat16)
    cb_bf = codebook.astype(jnp.bfloat16)
    indices = _argmin_indices(x2_bf, a, cb_bf).reshape(-1)

    n_tok = B * W * H
    counts = jnp.bincount(indices, length=CB)
    cf = counts.astype(jnp.float32)
    p = cf / float(n_tok)
    log_probs = jnp.log(jnp.maximum(cf, 1.0)) - jnp.log(float(n_tok))
    entropy = -jnp.sum(jnp.where(counts > 0, p * log_probs, 0.0))
    perplexity = jnp.exp(entropy)
    perplexity_loss = 1.0 / perplexity
    quantized = jnp.take(codebook, indices, axis=0).reshape(B, W, H, C)
    quantized = jnp.transpose(quantized, (0, 3, 1, 2))
    loss = BETA * perplexity_loss
    quantized = xq + jax.lax.stop_gradient(quantized - xq)
    return (quantized, perplexity_loss, loss)


# speedup vs baseline: 1.3041x; 1.1472x over previous
import functools

import jax
import jax.numpy as jnp
from jax import lax
from jax.experimental import pallas as pl
from jax.experimental.pallas import tpu as pltpu
from jax.experimental.pallas import tpu_sc as plsc

CB = 8192
DIM = 256
BETA = 0.25
TM = 1024
JB = 1024
NJ = CB // JB


def _dist_kernel(x_ref, a_ref, cb_ref, idx_ref, bestd_ref, besti_ref):
    j = pl.program_id(1)

    @pl.when(j == 0)
    def _init():
        bestd_ref[...] = jnp.full((TM, 1), jnp.inf, jnp.float32)
        besti_ref[...] = jnp.zeros((TM, 1), jnp.int32)

    s = jax.lax.dot_general(
        x_ref[...], cb_ref[...],
        dimension_numbers=(((1,), (1,)), ((), ())),
        preferred_element_type=jnp.float32,
    )
    u = a_ref[...] + float(DIM)
    d = u - s
    dmin = jnp.min(d, axis=1, keepdims=True)
    cols = jax.lax.broadcasted_iota(jnp.int32, (TM, JB), 1)
    cand = jnp.where(d == dmin, cols, jnp.int32(JB))
    imin = jnp.min(cand, axis=1, keepdims=True) + j * JB

    better = dmin < bestd_ref[...]
    tie = dmin == bestd_ref[...]
    besti_ref[...] = jnp.where(
        better, imin,
        jnp.where(tie, jnp.minimum(imin, besti_ref[...]), besti_ref[...]))
    nbd = jnp.where(better, dmin, bestd_ref[...])
    bestd_ref[...] = jnp.where(
        j == NJ // 2 - 1,
        nbd.astype(jnp.bfloat16).astype(jnp.float32), nbd)

    @pl.when(j == NJ - 1)
    def _out():
        idx_ref[...] = besti_ref[...]


def _argmin_indices(x2_bf, a, cb_bf):
    T = x2_bf.shape[0]
    return pl.pallas_call(
        _dist_kernel,
        grid=(T // TM, NJ),
        in_specs=[
            pl.BlockSpec((TM, DIM), lambda t, j: (t, 0)),
            pl.BlockSpec((TM, 1), lambda t, j: (t, 0)),
            pl.BlockSpec((JB, DIM), lambda t, j: (j, 0)),
        ],
        out_specs=pl.BlockSpec((TM, 1), lambda t, j: (t, 0)),
        out_shape=jax.ShapeDtypeStruct((T, 1), jnp.int32),
        scratch_shapes=[
            pltpu.VMEM((TM, 1), jnp.float32),
            pltpu.VMEM((TM, 1), jnp.int32),
        ],
        compiler_params=pltpu.CompilerParams(
            dimension_semantics=("parallel", "arbitrary")),
    )(x2_bf, a, cb_bf)


_NW = 32
_CHUNK = 128


def _sc_gather(codebook, indices):
    T = indices.shape[0]
    b_per_w = T // _NW
    mesh = plsc.VectorSubcoreMesh(core_axis_name="c", subcore_axis_name="s")

    @functools.partial(
        pl.kernel, mesh=mesh,
        out_type=jax.ShapeDtypeStruct((T, DIM), jnp.float32),
        scratch_types=[
            pltpu.VMEM((_CHUNK,), jnp.int32),
            pltpu.VMEM((_CHUNK, DIM), jnp.float32),
            pltpu.SemaphoreType.DMA,
        ],
    )
    def gather_k(table_hbm, idx_hbm, out_hbm, idx_v, rows_v, sem):
        wid = lax.axis_index("s") * 2 + lax.axis_index("c")
        base = wid * b_per_w
        for c in range(b_per_w // _CHUNK):
            off = base + c * _CHUNK
            pltpu.sync_copy(idx_hbm.at[pl.ds(off, _CHUNK)], idx_v)
            pltpu.async_copy(table_hbm.at[idx_v], rows_v, sem).wait()
            pltpu.sync_copy(rows_v, out_hbm.at[pl.ds(off, _CHUNK)])

    return gather_k(codebook, indices)


def kernel(x, codebook):
    B, C, W, H = x.shape
    xq = jax.nn.sigmoid(x * 100.0)
    xq = xq * 2.0 - 1.0
    flat_x = jnp.transpose(xq, (0, 2, 3, 1)).reshape(-1, C)
    a = jnp.sum(flat_x ** 2, axis=1, keepdims=True)
    x2_bf = (flat_x * 2.0).astype(jnp.bfloat16)
    cb_bf = codebook.astype(jnp.bfloat16)
    indices = _argmin_indices(x2_bf, a, cb_bf).reshape(-1)

    n_tok = B * W * H
    counts = jnp.bincount(indices, length=CB)
    cf = counts.astype(jnp.float32)
    p = cf / float(n_tok)
    log_probs = jnp.log(jnp.maximum(cf, 1.0)) - jnp.log(float(n_tok))
    entropy = -jnp.sum(jnp.where(counts > 0, p * log_probs, 0.0))
    perplexity = jnp.exp(entropy)
    perplexity_loss = 1.0 / perplexity
    quantized = _sc_gather(codebook, indices).reshape(B, W, H, C)
    quantized = jnp.transpose(quantized, (0, 3, 1, 2))
    loss = BETA * perplexity_loss
    quantized = xq + jax.lax.stop_gradient(quantized - xq)
    return (quantized, perplexity_loss, loss)
